# baseline (speedup 1.0000x reference)
import functools

import jax
import jax.numpy as jnp
from jax import lax
from jax.experimental import pallas as pl
from jax.experimental.pallas import tpu as pltpu
from jax.experimental.pallas import tpu_sc as plsc

BATCH = 16384
C = 1000
D = 256
ES = 10
MOM = 0.9
CM = 896
CT = C - CM

NC = 2
NS = 16
STRIP = 128
NSTRIP = CM // STRIP
NGRP = 4
GRP_ROWS = BATCH // NGRP
SCCH = 128
NSCCH = GRP_ROWS // SCCH


def _seg_sum_body(logits_hbm, idxs_hbm, zeros_hbm, out_hbm, idxs_v, buf_v, acc_v):
    cid = lax.axis_index("c")
    sid = lax.axis_index("s")
    w = sid * NC + cid
    st = w % NSTRIP
    g = w // NSTRIP

    @pl.when(w < NSTRIP * NGRP)
    def _():
        pltpu.sync_copy(zeros_hbm, acc_v)
        col0 = lax.broadcasted_iota(jnp.int32, (16,), 0)
        for ch in range(NSCCH):
            r0 = g * GRP_ROWS + ch * SCCH
            pltpu.sync_copy(
                logits_hbm.at[pl.ds(r0, SCCH), pl.ds(st * STRIP, STRIP)], buf_v)
            pltpu.sync_copy(idxs_hbm.at[pl.ds(r0, SCCH)], idxs_v)

            def body(j, carry):
                bvec = idxs_v[j, :]
                for gi in range(STRIP // 16):
                    xv = buf_v[j, pl.ds(gi * 16, 16)]
                    plsc.addupdate_scatter(acc_v, [bvec, col0 + (gi * 16)], xv)
                return carry

            lax.fori_loop(0, SCCH, body, 0)
        pltpu.sync_copy(acc_v, out_hbm.at[g, :, pl.ds(st * STRIP, STRIP)])


@functools.cache
def _seg_sum():
    mesh = plsc.VectorSubcoreMesh(core_axis_name="c", subcore_axis_name="s",
                                  num_cores=NC, num_subcores=NS)
    return pl.kernel(
        _seg_sum_body,
        out_type=jax.ShapeDtypeStruct((NGRP, D, CM), jnp.float32),
        mesh=mesh,
        compiler_params=pltpu.CompilerParams(needs_layout_passes=False),
        scratch_types=[
            pltpu.VMEM((SCCH, 16), jnp.int32),
            pltpu.VMEM((SCCH, STRIP), jnp.float32),
            pltpu.VMEM((D, STRIP), jnp.float32),
        ],
    )


BB = 1024
NB = BATCH // BB


def _dense_body(x_ref, lab_ref, idx_ref, acc_ref, lseb_ref, xtail_ref):
    i = pl.program_id(0)
    x = x_ref[...]
    m = jnp.max(x, axis=1, keepdims=True)
    t = x - m
    e = jnp.exp(t)
    s = jnp.sum(e, axis=1, keepdims=True)
    u = jnp.sum(t * e, axis=1, keepdims=True)
    logs = jnp.log(s)
    lse = m + logs
    lab = lab_ref[0, 0, :]
    col = lax.broadcasted_iota(jnp.int32, (BB, C), 1)
    xlab = jnp.sum(jnp.where(col == lab[:, None], x, 0.0), axis=1, keepdims=True)
    sum_ce = jnp.sum(lse - xlab)
    sum_plogp = jnp.sum(u / s - logs)
    idx = idx_ref[0, 0, :]
    r = lax.broadcasted_iota(jnp.int32, (BB, D), 1)
    onehot = (r == idx[:, None]).astype(jnp.float32)
    contrib = jnp.sum(onehot * lse, axis=0, keepdims=True)
    xt = x[:, CM:]
    xtail_blk = lax.dot_general(onehot, xt, (((0,), (0,)), ((), ())),
                                preferred_element_type=jnp.float32)
    lane = lax.broadcasted_iota(jnp.int32, (1, 128), 1)
    accv = jnp.where(lane == 0, sum_ce, 0.0) + jnp.where(lane == 1, sum_plogp, 0.0)

    @pl.when(i == 0)
    def _():
        acc_ref[...] = accv
        lseb_ref[...] = contrib
        xtail_ref[...] = xtail_blk

    @pl.when(i > 0)
    def _():
        acc_ref[...] += accv
        lseb_ref[...] += contrib
        xtail_ref[...] += xtail_blk


_dense = pl.pallas_call(
    _dense_body,
    grid=(NB,),
    in_specs=[
        pl.BlockSpec((BB, C), lambda i: (i, 0)),
        pl.BlockSpec((1, 1, BB), lambda i: (i, 0, 0)),
        pl.BlockSpec((1, 1, BB), lambda i: (i, 0, 0)),
    ],
    out_specs=[
        pl.BlockSpec((1, 128), lambda i: (0, 0)),
        pl.BlockSpec((1, D), lambda i: (0, 0)),
        pl.BlockSpec((D, CT), lambda i: (0, 0)),
    ],
    out_shape=[
        jax.ShapeDtypeStruct((1, 128), jnp.float32),
        jax.ShapeDtypeStruct((1, D), jnp.float32),
        jax.ShapeDtypeStruct((D, CT), jnp.float32),
    ],
)


def _combine_body(xp_ref, xtail_ref, sl_ref, lseb_ref, acc_ref, out_ref):
    Xm = xp_ref[0] + xp_ref[1] + xp_ref[2] + xp_ref[3]
    S = sl_ref[...]
    wsum = jnp.sum(S[:, :CM] * Xm) + jnp.sum(S[:, CM:] * xtail_ref[...])
    zs = jnp.sum(S, axis=1, keepdims=True)
    lsez = jnp.dot(lseb_ref[...], zs,
                   preferred_element_type=jnp.float32)[0, 0]
    sum_ce = acc_ref[0, 0]
    sum_plogp = acc_ref[0, 1]
    ce = sum_ce / BATCH
    selc = -(MOM * (wsum - lsez) + (1.0 - MOM) * sum_plogp) / BATCH
    lane = lax.broadcasted_iota(jnp.int32, (1, 2), 1)
    out_ref[...] = jnp.where(lane == 0, ce, selc)


_combine = pl.pallas_call(
    _combine_body,
    out_shape=jax.ShapeDtypeStruct((1, 2), jnp.float32),
)


def kernel(logits, labels, index, epoch, soft_labels):
    idx_splat = jnp.broadcast_to(index[:, None], (BATCH, 16))
    zeros = jnp.zeros((D, STRIP), jnp.float32)
    xacc = _seg_sum()(logits, idx_splat, zeros)
    labels3 = labels.reshape(NB, 1, BB)
    index3 = index.reshape(NB, 1, BB)
    acc, lseb, xtail = _dense(logits, labels3, index3)
    out = _combine(xacc, xtail, soft_labels, lseb, acc)
    return jnp.where(epoch <= ES, out[0, 0], out[0, 1])


# speedup vs baseline: 1.5084x; 1.5084x over previous
import functools

import jax
import jax.numpy as jnp
from jax import lax
from jax.experimental import pallas as pl
from jax.experimental.pallas import tpu as pltpu
from jax.experimental.pallas import tpu_sc as plsc

BATCH = 16384
C = 1000
D = 256
ES = 10
MOM = 0.9
CM = 896
CT = C - CM

NC = 2
NS = 16
STRIP = 128
NSTRIP = CM // STRIP
NGRP = 4
GRP_ROWS = BATCH // NGRP
SCCH = 256
NSCCH = GRP_ROWS // SCCH


def _seg_sum_body(logits_hbm, idxs_hbm, zeros_hbm, out_hbm, idxs_v, buf_v, acc_v):
    cid = lax.axis_index("c")
    sid = lax.axis_index("s")
    w = sid * NC + cid
    st = w % NSTRIP
    g = w // NSTRIP

    @pl.when(w < NSTRIP * NGRP)
    def _():
        pltpu.sync_copy(zeros_hbm, acc_v)
        col0 = lax.broadcasted_iota(jnp.int32, (16,), 0)
        for ch in range(NSCCH):
            r0 = g * GRP_ROWS + ch * SCCH
            pltpu.sync_copy(
                logits_hbm.at[pl.ds(r0, SCCH), pl.ds(st * STRIP, STRIP)], buf_v)
            pltpu.sync_copy(idxs_hbm.at[pl.ds(r0, SCCH)], idxs_v)

            @plsc.parallel_loop(0, SCCH, 1, unroll=8)
            def body(j):
                bvec = idxs_v[j, :]
                for gi in range(STRIP // 16):
                    xv = buf_v[j, pl.ds(gi * 16, 16)]
                    plsc.addupdate_scatter(acc_v, [bvec, col0 + (gi * 16)], xv)
        pltpu.sync_copy(acc_v, out_hbm.at[g, :, pl.ds(st * STRIP, STRIP)])


@functools.cache
def _seg_sum():
    mesh = plsc.VectorSubcoreMesh(core_axis_name="c", subcore_axis_name="s",
                                  num_cores=NC, num_subcores=NS)
    return pl.kernel(
        _seg_sum_body,
        out_type=jax.ShapeDtypeStruct((NGRP, D, CM), jnp.float32),
        mesh=mesh,
        compiler_params=pltpu.CompilerParams(needs_layout_passes=False),
        scratch_types=[
            pltpu.VMEM((SCCH, 16), jnp.int32),
            pltpu.VMEM((SCCH, STRIP), jnp.float32),
            pltpu.VMEM((D, STRIP), jnp.float32),
        ],
    )


BB = 1024
NB = BATCH // BB


def _dense_body(x_ref, lab_ref, idx_ref, acc_ref, lseb_ref, xtail_ref):
    i = pl.program_id(0)
    x = x_ref[...]
    m = jnp.max(x, axis=1, keepdims=True)
    t = x - m
    e = jnp.exp(t)
    s = jnp.sum(e, axis=1, keepdims=True)
    u = jnp.sum(t * e, axis=1, keepdims=True)
    logs = jnp.log(s)
    lse = m + logs
    lab = lab_ref[0, 0, :]
    col = lax.broadcasted_iota(jnp.int32, (BB, C), 1)
    xlab = jnp.sum(jnp.where(col == lab[:, None], x, 0.0), axis=1, keepdims=True)
    sum_ce = jnp.sum(lse - xlab)
    sum_plogp = jnp.sum(u / s - logs)
    idx = idx_ref[0, 0, :]
    r = lax.broadcasted_iota(jnp.int32, (BB, D), 1)
    onehot = (r == idx[:, None]).astype(jnp.float32)
    contrib = jnp.sum(onehot * lse, axis=0, keepdims=True)
    xt = x[:, CM:]
    xtail_blk = lax.dot_general(onehot, xt, (((0,), (0,)), ((), ())),
                                preferred_element_type=jnp.float32)
    lane = lax.broadcasted_iota(jnp.int32, (1, 128), 1)
    accv = jnp.where(lane == 0, sum_ce, 0.0) + jnp.where(lane == 1, sum_plogp, 0.0)

    @pl.when(i == 0)
    def _():
        acc_ref[...] = accv
        lseb_ref[...] = contrib
        xtail_ref[...] = xtail_blk

    @pl.when(i > 0)
    def _():
        acc_ref[...] += accv
        lseb_ref[...] += contrib
        xtail_ref[...] += xtail_blk


_dense = pl.pallas_call(
    _dense_body,
    grid=(NB,),
    in_specs=[
        pl.BlockSpec((BB, C), lambda i: (i, 0)),
        pl.BlockSpec((1, 1, BB), lambda i: (i, 0, 0)),
        pl.BlockSpec((1, 1, BB), lambda i: (i, 0, 0)),
    ],
    out_specs=[
        pl.BlockSpec((1, 128), lambda i: (0, 0)),
        pl.BlockSpec((1, D), lambda i: (0, 0)),
        pl.BlockSpec((D, CT), lambda i: (0, 0)),
    ],
    out_shape=[
        jax.ShapeDtypeStruct((1, 128), jnp.float32),
        jax.ShapeDtypeStruct((1, D), jnp.float32),
        jax.ShapeDtypeStruct((D, CT), jnp.float32),
    ],
)


def _combine_body(xp_ref, xtail_ref, sl_ref, lseb_ref, acc_ref, out_ref):
    Xm = xp_ref[0] + xp_ref[1] + xp_ref[2] + xp_ref[3]
    S = sl_ref[...]
    wsum = jnp.sum(S[:, :CM] * Xm) + jnp.sum(S[:, CM:] * xtail_ref[...])
    zs = jnp.sum(S, axis=1, keepdims=True)
    lsez = jnp.dot(lseb_ref[...], zs,
                   preferred_element_type=jnp.float32)[0, 0]
    sum_ce = acc_ref[0, 0]
    sum_plogp = acc_ref[0, 1]
    ce = sum_ce / BATCH
    selc = -(MOM * (wsum - lsez) + (1.0 - MOM) * sum_plogp) / BATCH
    lane = lax.broadcasted_iota(jnp.int32, (1, 2), 1)
    out_ref[...] = jnp.where(lane == 0, ce, selc)


_combine = pl.pallas_call(
    _combine_body,
    out_shape=jax.ShapeDtypeStruct((1, 2), jnp.float32),
)


def kernel(logits, labels, index, epoch, soft_labels):
    idx_splat = jnp.broadcast_to(index[:, None], (BATCH, 16))
    zeros = jnp.zeros((D, STRIP), jnp.float32)
    xacc = _seg_sum()(logits, idx_splat, zeros)
    labels3 = labels.reshape(NB, 1, BB)
    index3 = index.reshape(NB, 1, BB)
    acc, lseb, xtail = _dense(logits, labels3, index3)
    out = _combine(xacc, xtail, soft_labels, lseb, acc)
    return jnp.where(epoch <= ES, out[0, 0], out[0, 1])


# speedup vs baseline: 1.9237x; 1.2753x over previous
import functools

import jax
import jax.numpy as jnp
from jax import lax
from jax.experimental import pallas as pl
from jax.experimental.pallas import tpu as pltpu
from jax.experimental.pallas import tpu_sc as plsc

BATCH = 16384
C = 1000
D = 256
ES = 10
MOM = 0.9
CM = 896
CT = C - CM

NC = 2
NS = 16
STRIP = 128
NSTRIP = CM // STRIP
NGRP = 4
GRP_ROWS = BATCH // NGRP
SCCH = 128
NSCCH = GRP_ROWS // SCCH


def _seg_sum_body(logits_hbm, idxs_hbm, zeros_hbm, out_hbm,
                  xb0, xb1, ib0, ib1, acc_v, xs0, xs1, is0, is1):
    cid = lax.axis_index("c")
    sid = lax.axis_index("s")
    w = sid * NC + cid
    st = w % NSTRIP
    g = w // NSTRIP
    xbufs, ibufs = (xb0, xb1), (ib0, ib1)
    xsems, isems = (xs0, xs1), (is0, is1)

    @pl.when(w < NSTRIP * NGRP)
    def _():
        col0 = lax.broadcasted_iota(jnp.int32, (16,), 0)

        def start(ch, b):
            r0 = g * GRP_ROWS + ch * SCCH
            pltpu.async_copy(
                logits_hbm.at[pl.ds(r0, SCCH), pl.ds(st * STRIP, STRIP)],
                xbufs[b], xsems[b])
            pltpu.async_copy(idxs_hbm.at[pl.ds(r0, SCCH)], ibufs[b], isems[b])

        def wait(b):
            pltpu.make_async_copy(
                logits_hbm.at[pl.ds(0, SCCH), pl.ds(0, STRIP)],
                xbufs[b], xsems[b]).wait()
            pltpu.make_async_copy(idxs_hbm.at[pl.ds(0, SCCH)], ibufs[b],
                                  isems[b]).wait()

        def process(b):
            xbuf, ibuf = xbufs[b], ibufs[b]

            @plsc.parallel_loop(0, SCCH, 1, unroll=8)
            def body(j):
                bvec = ibuf[j, :]
                for gi in range(STRIP // 16):
                    xv = xbuf[j, pl.ds(gi * 16, 16)]
                    plsc.addupdate_scatter(acc_v, [bvec, col0 + (gi * 16)], xv)

        start(0, 0)
        pltpu.sync_copy(zeros_hbm, acc_v)

        def pbody(p, carry):
            ch0 = 2 * p
            start(ch0 + 1, 1)
            wait(0)
            process(0)

            @pl.when(p + 1 < NSCCH // 2)
            def _():
                start(ch0 + 2, 0)

            wait(1)
            process(1)
            return carry

        lax.fori_loop(0, NSCCH // 2, pbody, 0)
        pltpu.sync_copy(acc_v, out_hbm.at[g, :, pl.ds(st * STRIP, STRIP)])


@functools.cache
def _seg_sum():
    mesh = plsc.VectorSubcoreMesh(core_axis_name="c", subcore_axis_name="s",
                                  num_cores=NC, num_subcores=NS)
    return pl.kernel(
        _seg_sum_body,
        out_type=jax.ShapeDtypeStruct((NGRP, D, CM), jnp.float32),
        mesh=mesh,
        compiler_params=pltpu.CompilerParams(needs_layout_passes=False),
        scratch_types=[
            pltpu.VMEM((SCCH, STRIP), jnp.float32),
            pltpu.VMEM((SCCH, STRIP), jnp.float32),
            pltpu.VMEM((SCCH, 16), jnp.int32),
            pltpu.VMEM((SCCH, 16), jnp.int32),
            pltpu.VMEM((D, STRIP), jnp.float32),
            pltpu.SemaphoreType.DMA,
            pltpu.SemaphoreType.DMA,
            pltpu.SemaphoreType.DMA,
            pltpu.SemaphoreType.DMA,
        ],
    )


BB = 1024
NB = BATCH // BB


def _dense_body(x_ref, lab_ref, idx_ref, acc_ref, lseb_ref, xtail_ref):
    i = pl.program_id(0)
    x = x_ref[...]
    m = jnp.max(x, axis=1, keepdims=True)
    t = x - m
    e = jnp.exp(t)
    s = jnp.sum(e, axis=1, keepdims=True)
    u = jnp.sum(t * e, axis=1, keepdims=True)
    logs = jnp.log(s)
    lse = m + logs
    lab = lab_ref[0, 0, :]
    col = lax.broadcasted_iota(jnp.int32, (BB, C), 1)
    xlab = jnp.sum(jnp.where(col == lab[:, None], x, 0.0), axis=1, keepdims=True)
    sum_ce = jnp.sum(lse - xlab)
    sum_plogp = jnp.sum(u / s - logs)
    idx = idx_ref[0, 0, :]
    r = lax.broadcasted_iota(jnp.int32, (BB, D), 1)
    onehot = (r == idx[:, None]).astype(jnp.float32)
    contrib = jnp.sum(onehot * lse, axis=0, keepdims=True)
    xt = x[:, CM:]
    xtail_blk = lax.dot_general(onehot, xt, (((0,), (0,)), ((), ())),
                                preferred_element_type=jnp.float32)
    lane = lax.broadcasted_iota(jnp.int32, (1, 128), 1)
    accv = jnp.where(lane == 0, sum_ce, 0.0) + jnp.where(lane == 1, sum_plogp, 0.0)

    @pl.when(i == 0)
    def _():
        acc_ref[...] = accv
        lseb_ref[...] = contrib
        xtail_ref[...] = xtail_blk

    @pl.when(i > 0)
    def _():
        acc_ref[...] += accv
        lseb_ref[...] += contrib
        xtail_ref[...] += xtail_blk


_dense = pl.pallas_call(
    _dense_body,
    grid=(NB,),
    in_specs=[
        pl.BlockSpec((BB, C), lambda i: (i, 0)),
        pl.BlockSpec((1, 1, BB), lambda i: (i, 0, 0)),
        pl.BlockSpec((1, 1, BB), lambda i: (i, 0, 0)),
    ],
    out_specs=[
        pl.BlockSpec((1, 128), lambda i: (0, 0)),
        pl.BlockSpec((1, D), lambda i: (0, 0)),
        pl.BlockSpec((D, CT), lambda i: (0, 0)),
    ],
    out_shape=[
        jax.ShapeDtypeStruct((1, 128), jnp.float32),
        jax.ShapeDtypeStruct((1, D), jnp.float32),
        jax.ShapeDtypeStruct((D, CT), jnp.float32),
    ],
)


def _combine_body(xp_ref, xtail_ref, sl_ref, lseb_ref, acc_ref, out_ref):
    Xm = xp_ref[0] + xp_ref[1] + xp_ref[2] + xp_ref[3]
    S = sl_ref[...]
    wsum = jnp.sum(S[:, :CM] * Xm) + jnp.sum(S[:, CM:] * xtail_ref[...])
    zs = jnp.sum(S, axis=1, keepdims=True)
    lsez = jnp.dot(lseb_ref[...], zs,
                   preferred_element_type=jnp.float32)[0, 0]
    sum_ce = acc_ref[0, 0]
    sum_plogp = acc_ref[0, 1]
    ce = sum_ce / BATCH
    selc = -(MOM * (wsum - lsez) + (1.0 - MOM) * sum_plogp) / BATCH
    lane = lax.broadcasted_iota(jnp.int32, (1, 2), 1)
    out_ref[...] = jnp.where(lane == 0, ce, selc)


_combine = pl.pallas_call(
    _combine_body,
    out_shape=jax.ShapeDtypeStruct((1, 2), jnp.float32),
)


def kernel(logits, labels, index, epoch, soft_labels):
    idx_splat = jnp.broadcast_to(index[:, None], (BATCH, 16))
    zeros = jnp.zeros((D, STRIP), jnp.float32)
    xacc = _seg_sum()(logits, idx_splat, zeros)
    labels3 = labels.reshape(NB, 1, BB)
    index3 = index.reshape(NB, 1, BB)
    acc, lseb, xtail = _dense(logits, labels3, index3)
    out = _combine(xacc, xtail, soft_labels, lseb, acc)
    return jnp.where(epoch <= ES, out[0, 0], out[0, 1])


# speedup vs baseline: 1.9248x; 1.0006x over previous
import functools

import jax
import jax.numpy as jnp
from jax import lax
from jax.experimental import pallas as pl
from jax.experimental.pallas import tpu as pltpu
from jax.experimental.pallas import tpu_sc as plsc

BATCH = 16384
C = 1000
D = 256
ES = 10
MOM = 0.9
CM = 896
CT = C - CM

NC = 2
NS = 16
STRIP = 128
NSTRIP = CM // STRIP
NGRP = 4
GRP_ROWS = BATCH // NGRP
SCCH = 128
NSCCH = GRP_ROWS // SCCH


def _seg_sum_body(logits_hbm, idxs_hbm, zeros_hbm, out_hbm,
                  xb0, xb1, ib0, ib1, acc_v, xs0, xs1, is0, is1):
    cid = lax.axis_index("c")
    sid = lax.axis_index("s")
    w = sid * NC + cid
    st = w % NSTRIP
    g = w // NSTRIP
    xbufs, ibufs = (xb0, xb1), (ib0, ib1)
    xsems, isems = (xs0, xs1), (is0, is1)

    @pl.when(w < NSTRIP * NGRP)
    def _():
        col0 = lax.broadcasted_iota(jnp.int32, (16,), 0)

        def start(ch, b):
            r0 = g * GRP_ROWS + ch * SCCH
            pltpu.async_copy(
                logits_hbm.at[pl.ds(r0, SCCH), pl.ds(st * STRIP, STRIP)],
                xbufs[b], xsems[b])
            pltpu.async_copy(idxs_hbm.at[pl.ds(r0, SCCH)], ibufs[b], isems[b])

        def wait(b):
            pltpu.make_async_copy(
                logits_hbm.at[pl.ds(0, SCCH), pl.ds(0, STRIP)],
                xbufs[b], xsems[b]).wait()
            pltpu.make_async_copy(idxs_hbm.at[pl.ds(0, SCCH)], ibufs[b],
                                  isems[b]).wait()

        def process(b):
            xbuf, ibuf = xbufs[b], ibufs[b]

            @plsc.parallel_loop(0, SCCH, 1, unroll=8)
            def body(j):
                bvec = ibuf[j, :]
                for gi in range(STRIP // 16):
                    xv = xbuf[j, pl.ds(gi * 16, 16)]
                    plsc.addupdate_scatter(acc_v, [bvec, col0 + (gi * 16)], xv)

        start(0, 0)
        pltpu.sync_copy(zeros_hbm, acc_v)

        def pbody(p, carry):
            ch0 = 2 * p
            start(ch0 + 1, 1)
            wait(0)
            process(0)

            @pl.when(p + 1 < NSCCH // 2)
            def _():
                start(ch0 + 2, 0)

            wait(1)
            process(1)
            return carry

        lax.fori_loop(0, NSCCH // 2, pbody, 0)
        pltpu.sync_copy(acc_v, out_hbm.at[g, :, pl.ds(st * STRIP, STRIP)])


@functools.cache
def _seg_sum():
    mesh = plsc.VectorSubcoreMesh(core_axis_name="c", subcore_axis_name="s",
                                  num_cores=NC, num_subcores=NS)
    return pl.kernel(
        _seg_sum_body,
        out_type=jax.ShapeDtypeStruct((NGRP, D, CM), jnp.float32),
        mesh=mesh,
        compiler_params=pltpu.CompilerParams(needs_layout_passes=False,
                                             has_side_effects=False),
        scratch_types=[
            pltpu.VMEM((SCCH, STRIP), jnp.float32),
            pltpu.VMEM((SCCH, STRIP), jnp.float32),
            pltpu.VMEM((SCCH, 16), jnp.int32),
            pltpu.VMEM((SCCH, 16), jnp.int32),
            pltpu.VMEM((D, STRIP), jnp.float32),
            pltpu.SemaphoreType.DMA,
            pltpu.SemaphoreType.DMA,
            pltpu.SemaphoreType.DMA,
            pltpu.SemaphoreType.DMA,
        ],
    )


BB = 1024
NB = BATCH // BB


def _dense_body(x_ref, lab_ref, idx_ref, acc_ref, lseb_ref, xtail_ref):
    i = pl.program_id(0)
    x = x_ref[...]
    m = jnp.max(x, axis=1, keepdims=True)
    t = x - m
    e = jnp.exp(t)
    s = jnp.sum(e, axis=1, keepdims=True)
    u = jnp.sum(t * e, axis=1, keepdims=True)
    logs = jnp.log(s)
    lse = m + logs
    lab = lab_ref[0, 0, :]
    col = lax.broadcasted_iota(jnp.int32, (BB, C), 1)
    xlab = jnp.sum(jnp.where(col == lab[:, None], x, 0.0), axis=1, keepdims=True)
    sum_ce = jnp.sum(lse - xlab)
    sum_plogp = jnp.sum(u / s - logs)
    idx = idx_ref[0, 0, :]
    r = lax.broadcasted_iota(jnp.int32, (BB, D), 1)
    onehot = (r == idx[:, None]).astype(jnp.float32)
    contrib = jnp.sum(onehot * lse, axis=0, keepdims=True)
    xt = x[:, CM:]
    xtail_blk = lax.dot_general(onehot, xt, (((0,), (0,)), ((), ())),
                                preferred_element_type=jnp.float32)
    lane = lax.broadcasted_iota(jnp.int32, (1, 128), 1)
    accv = jnp.where(lane == 0, sum_ce, 0.0) + jnp.where(lane == 1, sum_plogp, 0.0)

    @pl.when(i == 0)
    def _():
        acc_ref[...] = accv
        lseb_ref[...] = contrib
        xtail_ref[...] = xtail_blk

    @pl.when(i > 0)
    def _():
        acc_ref[...] += accv
        lseb_ref[...] += contrib
        xtail_ref[...] += xtail_blk


_dense = pl.pallas_call(
    _dense_body,
    grid=(NB,),
    in_specs=[
        pl.BlockSpec((BB, C), lambda i: (i, 0)),
        pl.BlockSpec((1, 1, BB), lambda i: (i, 0, 0)),
        pl.BlockSpec((1, 1, BB), lambda i: (i, 0, 0)),
    ],
    out_specs=[
        pl.BlockSpec((1, 128), lambda i: (0, 0)),
        pl.BlockSpec((1, D), lambda i: (0, 0)),
        pl.BlockSpec((D, CT), lambda i: (0, 0)),
    ],
    out_shape=[
        jax.ShapeDtypeStruct((1, 128), jnp.float32),
        jax.ShapeDtypeStruct((1, D), jnp.float32),
        jax.ShapeDtypeStruct((D, CT), jnp.float32),
    ],
)


def _combine_body(xp_ref, xtail_ref, sl_ref, lseb_ref, acc_ref, out_ref):
    Xm = xp_ref[0] + xp_ref[1] + xp_ref[2] + xp_ref[3]
    S = sl_ref[...]
    wsum = jnp.sum(S[:, :CM] * Xm) + jnp.sum(S[:, CM:] * xtail_ref[...])
    zs = jnp.sum(S, axis=1, keepdims=True)
    lsez = jnp.dot(lseb_ref[...], zs,
                   preferred_element_type=jnp.float32)[0, 0]
    sum_ce = acc_ref[0, 0]
    sum_plogp = acc_ref[0, 1]
    ce = sum_ce / BATCH
    selc = -(MOM * (wsum - lsez) + (1.0 - MOM) * sum_plogp) / BATCH
    lane = lax.broadcasted_iota(jnp.int32, (1, 2), 1)
    out_ref[...] = jnp.where(lane == 0, ce, selc)


_combine = pl.pallas_call(
    _combine_body,
    out_shape=jax.ShapeDtypeStruct((1, 2), jnp.float32),
)


def kernel(logits, labels, index, epoch, soft_labels):
    idx_splat = jnp.broadcast_to(index[:, None], (BATCH, 16))
    zeros = jnp.zeros((D, STRIP), jnp.float32)
    xacc = _seg_sum()(logits, idx_splat, zeros)
    labels3 = labels.reshape(NB, 1, BB)
    index3 = index.reshape(NB, 1, BB)
    acc, lseb, xtail = _dense(logits, labels3, index3)
    out = _combine(xacc, xtail, soft_labels, lseb, acc)
    return jnp.where(epoch <= ES, out[0, 0], out[0, 1])


# speedup vs baseline: 1.9255x; 1.0004x over previous
import functools

import jax
import jax.numpy as jnp
from jax import lax
from jax.experimental import pallas as pl
from jax.experimental.pallas import tpu as pltpu
from jax.experimental.pallas import tpu_sc as plsc

BATCH = 16384
C = 1000
D = 256
ES = 10
MOM = 0.9
CM = 896
CT = C - CM

NC = 2
NS = 16
STRIP = 128
NSTRIP = CM // STRIP
NGRP = 4
GRP_ROWS = BATCH // NGRP
SCCH = 128
NSCCH = GRP_ROWS // SCCH


def _seg_sum_body(logits_hbm, idxs_hbm, zeros_hbm, out_hbm,
                  xb0, xb1, ib0, ib1, acc_v, xs0, xs1, is0, is1):
    cid = lax.axis_index("c")
    sid = lax.axis_index("s")
    w = sid * NC + cid
    st = w % NSTRIP
    g = w // NSTRIP
    xbufs, ibufs = (xb0, xb1), (ib0, ib1)
    xsems, isems = (xs0, xs1), (is0, is1)

    @pl.when(w < NSTRIP * NGRP)
    def _():
        col0 = lax.broadcasted_iota(jnp.int32, (16,), 0)

        def start(ch, b):
            r0 = g * GRP_ROWS + ch * SCCH
            pltpu.async_copy(
                logits_hbm.at[pl.ds(r0, SCCH), pl.ds(st * STRIP, STRIP)],
                xbufs[b], xsems[b])
            pltpu.async_copy(idxs_hbm.at[pl.ds(r0, SCCH)], ibufs[b], isems[b])

        def wait(b):
            pltpu.make_async_copy(
                logits_hbm.at[pl.ds(0, SCCH), pl.ds(0, STRIP)],
                xbufs[b], xsems[b]).wait()
            pltpu.make_async_copy(idxs_hbm.at[pl.ds(0, SCCH)], ibufs[b],
                                  isems[b]).wait()

        def process(b):
            xbuf, ibuf = xbufs[b], ibufs[b]

            @plsc.parallel_loop(0, SCCH, 1, unroll=8)
            def body(j):
                bvec = ibuf[j, :]
                for gi in range(STRIP // 16):
                    xv = xbuf[j, pl.ds(gi * 16, 16)]
                    plsc.addupdate_scatter(acc_v, [bvec, col0 + (gi * 16)], xv)

        start(0, 0)
        pltpu.sync_copy(zeros_hbm, acc_v)

        def pbody(p, carry):
            ch0 = 2 * p
            start(ch0 + 1, 1)
            wait(0)
            process(0)

            @pl.when(p + 1 < NSCCH // 2)
            def _():
                start(ch0 + 2, 0)

            wait(1)
            process(1)
            return carry

        lax.fori_loop(0, NSCCH // 2, pbody, 0)
        pltpu.sync_copy(acc_v, out_hbm.at[g, :, pl.ds(st * STRIP, STRIP)])


@functools.cache
def _seg_sum():
    mesh = plsc.VectorSubcoreMesh(core_axis_name="c", subcore_axis_name="s",
                                  num_cores=NC, num_subcores=NS)
    return pl.kernel(
        _seg_sum_body,
        out_type=jax.ShapeDtypeStruct((NGRP, D, CM), jnp.float32),
        mesh=mesh,
        compiler_params=pltpu.CompilerParams(needs_layout_passes=False,
                                             has_side_effects=False),
        cost_estimate=pl.CostEstimate(flops=14_680_064,
                                      bytes_accessed=62_914_560,
                                      transcendentals=0),
        scratch_types=[
            pltpu.VMEM((SCCH, STRIP), jnp.float32),
            pltpu.VMEM((SCCH, STRIP), jnp.float32),
            pltpu.VMEM((SCCH, 16), jnp.int32),
            pltpu.VMEM((SCCH, 16), jnp.int32),
            pltpu.VMEM((D, STRIP), jnp.float32),
            pltpu.SemaphoreType.DMA,
            pltpu.SemaphoreType.DMA,
            pltpu.SemaphoreType.DMA,
            pltpu.SemaphoreType.DMA,
        ],
    )


BB = 1024
NB = BATCH // BB


def _dense_body(x_ref, lab_ref, idx_ref, acc_ref, lseb_ref, xtail_ref):
    i = pl.program_id(0)
    x = x_ref[...]
    m = jnp.max(x, axis=1, keepdims=True)
    t = x - m
    e = jnp.exp(t)
    s = jnp.sum(e, axis=1, keepdims=True)
    u = jnp.sum(t * e, axis=1, keepdims=True)
    logs = jnp.log(s)
    lse = m + logs
    lab = lab_ref[0, 0, :]
    col = lax.broadcasted_iota(jnp.int32, (BB, C), 1)
    xlab = jnp.sum(jnp.where(col == lab[:, None], x, 0.0), axis=1, keepdims=True)
    sum_ce = jnp.sum(lse - xlab)
    sum_plogp = jnp.sum(u / s - logs)
    idx = idx_ref[0, 0, :]
    r = lax.broadcasted_iota(jnp.int32, (BB, D), 1)
    onehot = (r == idx[:, None]).astype(jnp.float32)
    contrib = jnp.sum(onehot * lse, axis=0, keepdims=True)
    xt = x[:, CM:]
    xtail_blk = lax.dot_general(onehot, xt, (((0,), (0,)), ((), ())),
                                preferred_element_type=jnp.float32)
    lane = lax.broadcasted_iota(jnp.int32, (1, 128), 1)
    accv = jnp.where(lane == 0, sum_ce, 0.0) + jnp.where(lane == 1, sum_plogp, 0.0)

    @pl.when(i == 0)
    def _():
        acc_ref[...] = accv
        lseb_ref[...] = contrib
        xtail_ref[...] = xtail_blk

    @pl.when(i > 0)
    def _():
        acc_ref[...] += accv
        lseb_ref[...] += contrib
        xtail_ref[...] += xtail_blk


_dense = pl.pallas_call(
    _dense_body,
    grid=(NB,),
    in_specs=[
        pl.BlockSpec((BB, C), lambda i: (i, 0)),
        pl.BlockSpec((1, 1, BB), lambda i: (i, 0, 0)),
        pl.BlockSpec((1, 1, BB), lambda i: (i, 0, 0)),
    ],
    out_specs=[
        pl.BlockSpec((1, 128), lambda i: (0, 0)),
        pl.BlockSpec((1, D), lambda i: (0, 0)),
        pl.BlockSpec((D, CT), lambda i: (0, 0)),
    ],
    out_shape=[
        jax.ShapeDtypeStruct((1, 128), jnp.float32),
        jax.ShapeDtypeStruct((1, D), jnp.float32),
        jax.ShapeDtypeStruct((D, CT), jnp.float32),
    ],
)


def _combine_body(xp_ref, xtail_ref, sl_ref, lseb_ref, acc_ref, out_ref):
    Xm = xp_ref[0] + xp_ref[1] + xp_ref[2] + xp_ref[3]
    S = sl_ref[...]
    wsum = jnp.sum(S[:, :CM] * Xm) + jnp.sum(S[:, CM:] * xtail_ref[...])
    zs = jnp.sum(S, axis=1, keepdims=True)
    lsez = jnp.dot(lseb_ref[...], zs,
                   preferred_element_type=jnp.float32)[0, 0]
    sum_ce = acc_ref[0, 0]
    sum_plogp = acc_ref[0, 1]
    ce = sum_ce / BATCH
    selc = -(MOM * (wsum - lsez) + (1.0 - MOM) * sum_plogp) / BATCH
    lane = lax.broadcasted_iota(jnp.int32, (1, 2), 1)
    out_ref[...] = jnp.where(lane == 0, ce, selc)


_combine = pl.pallas_call(
    _combine_body,
    out_shape=jax.ShapeDtypeStruct((1, 2), jnp.float32),
)


def kernel(logits, labels, index, epoch, soft_labels):
    idx_splat = jnp.broadcast_to(index[:, None], (BATCH, 16))
    zeros = jnp.zeros((D, STRIP), jnp.float32)
    xacc = _seg_sum()(logits, idx_splat, zeros)
    labels3 = labels.reshape(NB, 1, BB)
    index3 = index.reshape(NB, 1, BB)
    acc, lseb, xtail = _dense(logits, labels3, index3)
    out = _combine(xacc, xtail, soft_labels, lseb, acc)
    return jnp.where(epoch <= ES, out[0, 0], out[0, 1])


# speedup vs baseline: 2.6572x; 1.3800x over previous
import functools

import jax
import jax.numpy as jnp
from jax import lax
from jax.experimental import pallas as pl
from jax.experimental.pallas import tpu as pltpu
from jax.experimental.pallas import tpu_sc as plsc

BATCH = 16384
C = 1000
D = 256
ES = 10
MOM = 0.9
CM = 0
CT = C - CM

NC = 2
NS = 16
STRIP = 128
NSTRIP = max(CM // STRIP, 1)
NGRP = 4
GRP_ROWS = BATCH // NGRP
SCCH = 128
NSCCH = GRP_ROWS // SCCH


def _seg_sum_body(logits_hbm, idxs_hbm, zeros_hbm, out_hbm,
                  xb0, xb1, ib0, ib1, acc_v, xs0, xs1, is0, is1):
    cid = lax.axis_index("c")
    sid = lax.axis_index("s")
    w = sid * NC + cid
    st = w % NSTRIP
    g = w // NSTRIP
    xbufs, ibufs = (xb0, xb1), (ib0, ib1)
    xsems, isems = (xs0, xs1), (is0, is1)

    @pl.when(w < NSTRIP * NGRP)
    def _():
        col0 = lax.broadcasted_iota(jnp.int32, (16,), 0)

        def start(ch, b):
            r0 = g * GRP_ROWS + ch * SCCH
            pltpu.async_copy(
                logits_hbm.at[pl.ds(r0, SCCH), pl.ds(st * STRIP, STRIP)],
                xbufs[b], xsems[b])
            pltpu.async_copy(idxs_hbm.at[pl.ds(r0, SCCH)], ibufs[b], isems[b])

        def wait(b):
            pltpu.make_async_copy(
                logits_hbm.at[pl.ds(0, SCCH), pl.ds(0, STRIP)],
                xbufs[b], xsems[b]).wait()
            pltpu.make_async_copy(idxs_hbm.at[pl.ds(0, SCCH)], ibufs[b],
                                  isems[b]).wait()

        def process(b):
            xbuf, ibuf = xbufs[b], ibufs[b]

            @plsc.parallel_loop(0, SCCH, 1, unroll=8)
            def body(j):
                bvec = ibuf[j, :]
                for gi in range(STRIP // 16):
                    xv = xbuf[j, pl.ds(gi * 16, 16)]
                    plsc.addupdate_scatter(acc_v, [bvec, col0 + (gi * 16)], xv)

        start(0, 0)
        pltpu.sync_copy(zeros_hbm, acc_v)

        def pbody(p, carry):
            ch0 = 2 * p
            start(ch0 + 1, 1)
            wait(0)
            process(0)

            @pl.when(p + 1 < NSCCH // 2)
            def _():
                start(ch0 + 2, 0)

            wait(1)
            process(1)
            return carry

        lax.fori_loop(0, NSCCH // 2, pbody, 0)
        pltpu.sync_copy(acc_v, out_hbm.at[g, :, pl.ds(st * STRIP, STRIP)])


@functools.cache
def _seg_sum():
    mesh = plsc.VectorSubcoreMesh(core_axis_name="c", subcore_axis_name="s",
                                  num_cores=NC, num_subcores=NS)
    return pl.kernel(
        _seg_sum_body,
        out_type=jax.ShapeDtypeStruct((NGRP, D, CM), jnp.float32),
        mesh=mesh,
        compiler_params=pltpu.CompilerParams(needs_layout_passes=False,
                                             has_side_effects=False),
        cost_estimate=pl.CostEstimate(flops=14_680_064,
                                      bytes_accessed=62_914_560,
                                      transcendentals=0),
        scratch_types=[
            pltpu.VMEM((SCCH, STRIP), jnp.float32),
            pltpu.VMEM((SCCH, STRIP), jnp.float32),
            pltpu.VMEM((SCCH, 16), jnp.int32),
            pltpu.VMEM((SCCH, 16), jnp.int32),
            pltpu.VMEM((D, STRIP), jnp.float32),
            pltpu.SemaphoreType.DMA,
            pltpu.SemaphoreType.DMA,
            pltpu.SemaphoreType.DMA,
            pltpu.SemaphoreType.DMA,
        ],
    )


BB = 1024
NB = BATCH // BB


def _dense_body(x_ref, lab_ref, idx_ref, acc_ref, lseb_ref, xtail_ref):
    i = pl.program_id(0)
    x = x_ref[...]
    m = jnp.max(x, axis=1, keepdims=True)
    t = x - m
    e = jnp.exp(t)
    s = jnp.sum(e, axis=1, keepdims=True)
    u = jnp.sum(t * e, axis=1, keepdims=True)
    logs = jnp.log(s)
    lse = m + logs
    lab = lab_ref[0, 0, :]
    col = lax.broadcasted_iota(jnp.int32, (BB, C), 1)
    xlab = jnp.sum(jnp.where(col == lab[:, None], x, 0.0), axis=1, keepdims=True)
    sum_ce = jnp.sum(lse - xlab)
    sum_plogp = jnp.sum(u / s - logs)
    idx = idx_ref[0, 0, :]
    r = lax.broadcasted_iota(jnp.int32, (BB, D), 1)
    onehot = (r == idx[:, None]).astype(jnp.float32)
    contrib = jnp.sum(onehot * lse, axis=0, keepdims=True)
    xt = x[:, CM:].astype(jnp.bfloat16)
    xtail_blk = lax.dot_general(onehot.astype(jnp.bfloat16), xt,
                                (((0,), (0,)), ((), ())),
                                preferred_element_type=jnp.float32)
    lane = lax.broadcasted_iota(jnp.int32, (1, 128), 1)
    accv = jnp.where(lane == 0, sum_ce, 0.0) + jnp.where(lane == 1, sum_plogp, 0.0)

    @pl.when(i == 0)
    def _():
        acc_ref[...] = accv
        lseb_ref[...] = contrib
        xtail_ref[...] = xtail_blk

    @pl.when(i > 0)
    def _():
        acc_ref[...] += accv
        lseb_ref[...] += contrib
        xtail_ref[...] += xtail_blk


_dense = pl.pallas_call(
    _dense_body,
    grid=(NB,),
    in_specs=[
        pl.BlockSpec((BB, C), lambda i: (i, 0)),
        pl.BlockSpec((1, 1, BB), lambda i: (i, 0, 0)),
        pl.BlockSpec((1, 1, BB), lambda i: (i, 0, 0)),
    ],
    out_specs=[
        pl.BlockSpec((1, 128), lambda i: (0, 0)),
        pl.BlockSpec((1, D), lambda i: (0, 0)),
        pl.BlockSpec((D, CT), lambda i: (0, 0)),
    ],
    out_shape=[
        jax.ShapeDtypeStruct((1, 128), jnp.float32),
        jax.ShapeDtypeStruct((1, D), jnp.float32),
        jax.ShapeDtypeStruct((D, CT), jnp.float32),
    ],
)


def _combine_body(xtail_ref, sl_ref, lseb_ref, acc_ref, out_ref):
    S = sl_ref[...]
    wsum = jnp.sum(S[:, CM:] * xtail_ref[...])
    zs = jnp.sum(S, axis=1, keepdims=True)
    lsez = jnp.dot(lseb_ref[...], zs,
                   preferred_element_type=jnp.float32)[0, 0]
    sum_ce = acc_ref[0, 0]
    sum_plogp = acc_ref[0, 1]
    ce = sum_ce / BATCH
    selc = -(MOM * (wsum - lsez) + (1.0 - MOM) * sum_plogp) / BATCH
    lane = lax.broadcasted_iota(jnp.int32, (1, 2), 1)
    out_ref[...] = jnp.where(lane == 0, ce, selc)


_combine = pl.pallas_call(
    _combine_body,
    out_shape=jax.ShapeDtypeStruct((1, 2), jnp.float32),
)


def kernel(logits, labels, index, epoch, soft_labels):
    labels3 = labels.reshape(NB, 1, BB)
    index3 = index.reshape(NB, 1, BB)
    acc, lseb, xtail = _dense(logits, labels3, index3)
    out = _combine(xtail, soft_labels, lseb, acc)
    return jnp.where(epoch <= ES, out[0, 0], out[0, 1])
